# Initial kernel scaffold; baseline (speedup 1.0000x reference)
import jax
import jax.numpy as jnp
from jax.experimental import pallas as pl


def kernel(x, edge_index, W1, b1, W2, b2, Wc, bc):
    raise NotImplementedError("write your pallas kernel here")


# speedup vs baseline: 10.1559x; 10.1559x over previous
import functools

import jax
import jax.numpy as jnp
from jax import lax
from jax.experimental import pallas as pl
from jax.experimental.pallas import tpu as pltpu
from jax.experimental.pallas import tpu_sc as plsc

N = 10000
NP = 10240
E = 320000
NC, NS = 2, 16
NW = NC * NS
EPT = 10240
EP = NW * EPT
CHUNK = 128
NCHUNK = EPT // CHUNK
RPT = NP // NS
BLK = 1024
GRID = NP // BLK
DROW = NP // 128
FROW = NP * 2 // 128

_mesh = plsc.VectorSubcoreMesh(core_axis_name="c", subcore_axis_name="s",
                               num_cores=NC, num_subcores=NS)
_sc_params = pltpu.CompilerParams(needs_layout_passes=False)


def _zero_fill(ref, nrows, ncols):
    z = jnp.zeros((16,), jnp.float32)
    per_row = ncols // 16

    def body(i, _):
        ref[i // per_row, pl.ds((i % per_row) * 16, 16)] = z
        return 0

    lax.fori_loop(0, nrows * per_row, body, 0)


def _fill_iota_row(ref, n):
    base = lax.iota(jnp.int32, 16)
    for k in range(n // 16):
        ref[0, pl.ds(k * 16, 16)] = base + (k * 16)


def _repack(flat_ref, mat_ref, nrows):
    def body(r, _):
        for k in range(8):
            mat_ref[r, pl.ds(k * 16, 16)] = flat_ref[pl.ds(r * 128 + k * 16, 16)]
        return 0

    lax.fori_loop(0, nrows, body, 0)


@functools.partial(
    pl.kernel,
    out_type=jax.ShapeDtypeStruct((NC, DROW, 128), jnp.float32),
    mesh=_mesh,
    compiler_params=_sc_params,
    scratch_types=[
        pltpu.VMEM((CHUNK,), jnp.int32),
        pltpu.VMEM((NP,), jnp.float32),
        pltpu.VMEM((DROW, 128), jnp.float32),
        pltpu.VMEM((1, DROW), jnp.int32),
        pltpu.VMEM_SHARED((DROW, 128), jnp.float32),
    ],
)
def _deg_kernel(dst_hbm, out_hbm, dst_v, hist_v, mat_v, id_v, acc):
    cid = lax.axis_index("c")
    sid = lax.axis_index("s")
    wid = cid * NS + sid

    zero = jnp.zeros((16,), jnp.float32)

    def zb(i, _):
        hist_v[pl.ds(i * 16, 16)] = zero
        return 0

    lax.fori_loop(0, NP // 16, zb, 0)
    _zero_fill(mat_v, DROW, 128)
    _fill_iota_row(id_v, DROW)

    @pl.when(sid < DROW // 16)
    def _():
        pltpu.sync_copy(mat_v.at[pl.ds(sid * 16, 16)],
                        acc.at[pl.ds(sid * 16, 16)])
    plsc.subcore_barrier()

    one = jnp.ones((16,), jnp.float32)

    def body(j, _):
        pltpu.sync_copy(dst_hbm.at[pl.ds(wid * EPT + j * CHUNK, CHUNK)], dst_v)
        for k in range(CHUNK // 16):
            plsc.addupdate_scatter(hist_v, [dst_v[pl.ds(k * 16, 16)]], one)
        return 0

    lax.fori_loop(0, NCHUNK, body, 0)
    _repack(hist_v, mat_v, DROW)
    pltpu.sync_copy(mat_v, acc.at[id_v.at[0]], add=True)
    plsc.subcore_barrier()

    @pl.when(sid < DROW // 16)
    def _():
        pltpu.sync_copy(acc.at[pl.ds(sid * 16, 16)],
                        out_hbm.at[cid, pl.ds(sid * 16, 16)])


@functools.partial(
    pl.kernel,
    out_type=jax.ShapeDtypeStruct((NC, NP, 128), jnp.float32),
    mesh=_mesh,
    compiler_params=_sc_params,
    scratch_types=[
        pltpu.VMEM((CHUNK,), jnp.int32),
        pltpu.VMEM((1, CHUNK), jnp.int32),
        pltpu.VMEM((CHUNK, 128), jnp.float32),
        pltpu.VMEM_SHARED((NP, 128), jnp.float32),
        pltpu.SemaphoreType.DMA,
    ],
)
def _agg128(table_hbm, src_hbm, dst_hbm, out_hbm, src_v, dst_v, rows_v, acc, sem):
    cid = lax.axis_index("c")
    sid = lax.axis_index("s")
    wid = cid * NS + sid

    _zero_fill(rows_v, CHUNK, 128)
    for k in range(RPT // CHUNK):
        pltpu.sync_copy(rows_v, acc.at[pl.ds(sid * RPT + k * CHUNK, CHUNK)])
    plsc.subcore_barrier()

    def body(j, _):
        e0 = wid * EPT + j * CHUNK
        pltpu.sync_copy(src_hbm.at[pl.ds(e0, CHUNK)], src_v)
        pltpu.sync_copy(dst_hbm.at[pl.ds(e0, CHUNK)], dst_v.at[0])
        pltpu.async_copy(table_hbm.at[src_v], rows_v, sem).wait()
        pltpu.sync_copy(rows_v, acc.at[dst_v.at[0]], add=True)
        return 0

    lax.fori_loop(0, NCHUNK, body, 0)
    plsc.subcore_barrier()
    pltpu.sync_copy(acc.at[pl.ds(sid * RPT, RPT)],
                    out_hbm.at[cid, pl.ds(sid * RPT, RPT)])


@functools.partial(
    pl.kernel,
    out_type=jax.ShapeDtypeStruct((NC, FROW, 128), jnp.float32),
    mesh=_mesh,
    compiler_params=_sc_params,
    scratch_types=[
        pltpu.VMEM((CHUNK,), jnp.int32),
        pltpu.VMEM((CHUNK,), jnp.int32),
        pltpu.VMEM((NP * 2,), jnp.float32),
        pltpu.VMEM((NP * 2,), jnp.float32),
        pltpu.VMEM((FROW, 128), jnp.float32),
        pltpu.VMEM((1, FROW), jnp.int32),
        pltpu.VMEM_SHARED((FROW, 128), jnp.float32),
    ],
)
def _agg2(table_hbm, src_hbm, dst_hbm, out_hbm,
          src_v, dst_v, tab_v, acc_v, mat_v, id_v, acc):
    cid = lax.axis_index("c")
    sid = lax.axis_index("s")
    wid = cid * NS + sid

    zero = jnp.zeros((16,), jnp.float32)

    def zb(i, _):
        acc_v[pl.ds(i * 16, 16)] = zero
        return 0

    lax.fori_loop(0, NP * 2 // 16, zb, 0)
    _zero_fill(mat_v, FROW, 128)
    _fill_iota_row(id_v, FROW)

    @pl.when(sid < FROW // 16)
    def _():
        pltpu.sync_copy(mat_v.at[pl.ds(sid * 16, 16)],
                        acc.at[pl.ds(sid * 16, 16)])
    pltpu.sync_copy(table_hbm, tab_v)
    plsc.subcore_barrier()

    def body(j, _):
        e0 = wid * EPT + j * CHUNK
        pltpu.sync_copy(src_hbm.at[pl.ds(e0, CHUNK)], src_v)
        pltpu.sync_copy(dst_hbm.at[pl.ds(e0, CHUNK)], dst_v)
        for k in range(CHUNK // 16):
            s2 = dst_v[pl.ds(k * 16, 16)] * 2
            g2 = src_v[pl.ds(k * 16, 16)] * 2
            for c in range(2):
                v = plsc.load_gather(tab_v, [g2 + c])
                plsc.addupdate_scatter(acc_v, [s2 + c], v)
        return 0

    lax.fori_loop(0, NCHUNK, body, 0)
    _repack(acc_v, mat_v, FROW)
    pltpu.sync_copy(mat_v, acc.at[id_v.at[0]], add=True)
    plsc.subcore_barrier()

    @pl.when(sid < FROW // 16)
    def _():
        pltpu.sync_copy(acc.at[pl.ds(sid * 16, 16)],
                        out_hbm.at[cid, pl.ds(sid * 16, 16)])


def _dinv(deg_ref):
    return lax.rsqrt(deg_ref[0] + deg_ref[1] + 1.0)


def _tc_scale_in(x_ref, w1_ref, deg_ref, hs_ref):
    dinv = _dinv(deg_ref)
    h = jnp.dot(x_ref[...], w1_ref[...], preferred_element_type=jnp.float32)
    hs_ref[...] = h * dinv


def _tc_mid(p_ref, hs_ref, deg_ref, b1_ref, w2_ref, h2s_ref):
    dinv = _dinv(deg_ref)
    t1 = jnp.tanh((p_ref[0] + p_ref[1] + hs_ref[...]) * dinv + b1_ref[...])
    h2 = jnp.dot(t1, w2_ref[...], preferred_element_type=jnp.float32)
    h2s_ref[...] = h2 * dinv


def _tc_head(q_ref, h2s_ref, deg_ref, b2_ref, wc_ref, bc_ref, out_ref):
    dinv = _dinv(deg_ref)
    emb = jnp.tanh((q_ref[0] + q_ref[1] + h2s_ref[...]) * dinv + b2_ref[...])
    o = jnp.dot(emb, wc_ref[...], preferred_element_type=jnp.float32)
    out_ref[...] = jax.nn.sigmoid(o + bc_ref[...])


def _row_spec(d):
    return pl.BlockSpec((BLK, d), lambda i: (i, 0))


def _full_spec(shape):
    return pl.BlockSpec(shape, lambda i: (0,) * len(shape))


_deg_spec = pl.BlockSpec((NC, BLK, 1), lambda i: (0, i, 0))


def kernel(x, edge_index, W1, b1, W2, b2, Wc, bc):
    f32 = jnp.float32
    src = edge_index[0].astype(jnp.int32)
    dst = edge_index[1].astype(jnp.int32)
    pad = jnp.full((EP - E,), N, jnp.int32)
    src = jnp.concatenate([src, pad])
    dst = jnp.concatenate([dst, pad])
    xp = jnp.pad(x, ((0, NP - N), (0, 0)))
    b1r = b1.reshape(1, 128)
    b2r = b2.reshape(1, 2)
    bcr = bc.reshape(1, 1)

    deg = _deg_kernel(dst).reshape(NC, NP, 1)

    hs = pl.pallas_call(
        _tc_scale_in,
        grid=(GRID,),
        in_specs=[_row_spec(165), _full_spec((165, 128)), _deg_spec],
        out_specs=_row_spec(128),
        out_shape=jax.ShapeDtypeStruct((NP, 128), f32),
    )(xp, W1, deg)

    p = _agg128(hs, src, dst)

    h2s = pl.pallas_call(
        _tc_mid,
        grid=(GRID,),
        in_specs=[pl.BlockSpec((NC, BLK, 128), lambda i: (0, i, 0)),
                  _row_spec(128), _deg_spec,
                  _full_spec((1, 128)), _full_spec((128, 2))],
        out_specs=_row_spec(2),
        out_shape=jax.ShapeDtypeStruct((NP, 2), f32),
    )(p, hs, deg, b1r, W2)

    q = _agg2(h2s.reshape(NP * 2), src, dst).reshape(NC, NP, 2)

    out = pl.pallas_call(
        _tc_head,
        grid=(GRID,),
        in_specs=[pl.BlockSpec((NC, BLK, 2), lambda i: (0, i, 0)),
                  _row_spec(2), _deg_spec,
                  _full_spec((1, 2)), _full_spec((2, 1)), _full_spec((1, 1))],
        out_specs=_row_spec(1),
        out_shape=jax.ShapeDtypeStruct((NP, 1), f32),
    )(q, h2s, deg, b2r, Wc, bcr)

    return out[:N]


# speedup vs baseline: 29.7145x; 2.9258x over previous
import functools

import jax
import jax.numpy as jnp
from jax import lax
from jax.experimental import pallas as pl
from jax.experimental.pallas import tpu as pltpu
from jax.experimental.pallas import tpu_sc as plsc

N = 10000
NP = 10240
E = 320000
NC, NS = 2, 16
NW = NC * NS
EPT = E // NW
CHUNK = 80
NCHUNK = EPT // CHUNK
BLK = 1000
GRID = N // BLK
DROW = NP // 128
FROW = NP * 2 // 128

_mesh = plsc.VectorSubcoreMesh(core_axis_name="c", subcore_axis_name="s",
                               num_cores=NC, num_subcores=NS)
_sc_params = pltpu.CompilerParams(needs_layout_passes=False)


def _zero_fill(ref, nrows, ncols):
    z = jnp.zeros((16,), jnp.float32)
    per_row = ncols // 16

    def body(i, _):
        ref[i // per_row, pl.ds((i % per_row) * 16, 16)] = z
        return 0

    lax.fori_loop(0, nrows * per_row, body, 0)


def _zero_fill1(ref, n):
    z = jnp.zeros((16,), jnp.float32)

    def body(i, _):
        ref[pl.ds(i * 16, 16)] = z
        return 0

    lax.fori_loop(0, n // 16, body, 0)


def _fill_iota_row(ref, n):
    base = lax.iota(jnp.int32, 16)
    for k in range(n // 16):
        ref[0, pl.ds(k * 16, 16)] = base + (k * 16)


def _repack(flat_ref, mat_ref, nrows):
    def body(r, _):
        for k in range(8):
            mat_ref[r, pl.ds(k * 16, 16)] = flat_ref[pl.ds(r * 128 + k * 16, 16)]
        return 0

    lax.fori_loop(0, nrows, body, 0)


@functools.partial(
    pl.kernel,
    out_type=jax.ShapeDtypeStruct((NC, DROW, 128), jnp.float32),
    mesh=_mesh,
    compiler_params=_sc_params,
    scratch_types=[
        pltpu.VMEM((EPT,), jnp.int32),
        pltpu.VMEM((NP,), jnp.float32),
        pltpu.VMEM((DROW, 128), jnp.float32),
        pltpu.VMEM((1, DROW), jnp.int32),
        pltpu.VMEM_SHARED((DROW, 128), jnp.float32),
    ],
)
def _deg_kernel(dst_hbm, out_hbm, dst_v, hist_v, mat_v, id_v, acc):
    cid = lax.axis_index("c")
    sid = lax.axis_index("s")
    wid = cid * NS + sid

    pltpu.sync_copy(dst_hbm.at[pl.ds(wid * EPT, EPT)], dst_v)
    _zero_fill1(hist_v, NP)
    _zero_fill(mat_v, DROW, 128)
    _fill_iota_row(id_v, DROW)

    @pl.when(sid < DROW // 16)
    def _():
        pltpu.sync_copy(mat_v.at[pl.ds(sid * 16, 16)],
                        acc.at[pl.ds(sid * 16, 16)])
    plsc.subcore_barrier()

    one = jnp.ones((16,), jnp.float32)

    def body(i, _):
        plsc.addupdate_scatter(hist_v, [dst_v[pl.ds(i * 16, 16)]], one)
        return 0

    lax.fori_loop(0, EPT // 16, body, 0)
    _repack(hist_v, mat_v, DROW)
    pltpu.sync_copy(mat_v, acc.at[id_v.at[0]], add=True)
    plsc.subcore_barrier()

    @pl.when(sid < DROW // 16)
    def _():
        pltpu.sync_copy(acc.at[pl.ds(sid * 16, 16)],
                        out_hbm.at[cid, pl.ds(sid * 16, 16)])


@functools.partial(
    pl.kernel,
    out_type=jax.ShapeDtypeStruct((NC, N, 128), jnp.float32),
    mesh=_mesh,
    compiler_params=_sc_params,
    scratch_types=[
        pltpu.VMEM((EPT,), jnp.int32),
        pltpu.VMEM((2, CHUNK), jnp.int32),
        pltpu.VMEM((2, CHUNK, 128), jnp.float32),
        pltpu.VMEM_SHARED((NP, 128), jnp.float32),
        pltpu.SemaphoreType.DMA((2,)),
        pltpu.SemaphoreType.DMA((2,)),
        pltpu.SemaphoreType.DMA((2,)),
    ],
)
def _agg128(table_hbm, src_hbm, dst_hbm, out_hbm,
            src_v, dst_v, rows_v, acc, gsem, ssem, dsem):
    cid = lax.axis_index("c")
    sid = lax.axis_index("s")
    wid = cid * NS + sid
    ebase = wid * EPT

    pltpu.sync_copy(src_hbm.at[pl.ds(ebase, EPT)], src_v)
    for b in range(2):
        _zero_fill(rows_v.at[b], CHUNK, 128)
    for k in range(640 // CHUNK):
        pltpu.sync_copy(rows_v.at[0], acc.at[pl.ds(sid * 640 + k * CHUNK, CHUNK)])
    plsc.subcore_barrier()

    def dload(j, b):
        return pltpu.make_async_copy(
            dst_hbm.at[pl.ds(ebase + j * CHUNK, CHUNK)], dst_v.at[b], dsem.at[b])

    def gath(j, b):
        return pltpu.make_async_copy(
            table_hbm.at[src_v.at[pl.ds(j * CHUNK, CHUNK)]], rows_v.at[b],
            gsem.at[b])

    def scat(b):
        return pltpu.make_async_copy(
            rows_v.at[b], acc.at[dst_v.at[b]], ssem.at[b])

    dload(0, 0).start()
    gath(0, 0).start()

    def body(j, _):
        b = j & 1
        nb = 1 - b
        gath(j, b).wait()

        @pl.when(j >= 1)
        def _():
            scat(nb).wait()

        @pl.when(j + 1 < NCHUNK)
        def _():
            dload(j + 1, nb).start()
            gath(j + 1, nb).start()

        dload(j, b).wait()
        pltpu.async_copy(rows_v.at[b], acc.at[dst_v.at[b]], ssem.at[b],
                         add=True)
        return 0

    lax.fori_loop(0, NCHUNK, body, 0)
    scat((NCHUNK - 1) & 1).wait()
    plsc.subcore_barrier()

    @pl.when(sid < 10)
    def _():
        pltpu.sync_copy(acc.at[pl.ds(sid * BLK, BLK)],
                        out_hbm.at[cid, pl.ds(sid * BLK, BLK)])


@functools.partial(
    pl.kernel,
    out_type=jax.ShapeDtypeStruct((NC, FROW, 128), jnp.float32),
    mesh=_mesh,
    compiler_params=_sc_params,
    scratch_types=[
        pltpu.VMEM((EPT,), jnp.int32),
        pltpu.VMEM((EPT,), jnp.int32),
        pltpu.VMEM((N * 2,), jnp.float32),
        pltpu.VMEM((NP * 2,), jnp.float32),
        pltpu.VMEM((FROW, 128), jnp.float32),
        pltpu.VMEM((1, FROW), jnp.int32),
        pltpu.VMEM_SHARED((FROW, 128), jnp.float32),
    ],
)
def _agg2(table_hbm, src_hbm, dst_hbm, out_hbm,
          src_v, dst_v, tab_v, acc_v, mat_v, id_v, acc):
    cid = lax.axis_index("c")
    sid = lax.axis_index("s")
    wid = cid * NS + sid

    pltpu.sync_copy(src_hbm.at[pl.ds(wid * EPT, EPT)], src_v)
    pltpu.sync_copy(dst_hbm.at[pl.ds(wid * EPT, EPT)], dst_v)
    pltpu.sync_copy(table_hbm, tab_v)
    _zero_fill1(acc_v, NP * 2)
    _zero_fill(mat_v, FROW, 128)
    _fill_iota_row(id_v, FROW)

    @pl.when(sid < FROW // 16)
    def _():
        pltpu.sync_copy(mat_v.at[pl.ds(sid * 16, 16)],
                        acc.at[pl.ds(sid * 16, 16)])
    plsc.subcore_barrier()

    def body(i, _):
        s2 = src_v[pl.ds(i * 16, 16)] * 2
        d2 = dst_v[pl.ds(i * 16, 16)] * 2
        for c in range(2):
            v = plsc.load_gather(tab_v, [s2 + c])
            plsc.addupdate_scatter(acc_v, [d2 + c], v)
        return 0

    lax.fori_loop(0, EPT // 16, body, 0)
    _repack(acc_v, mat_v, FROW)
    pltpu.sync_copy(mat_v, acc.at[id_v.at[0]], add=True)
    plsc.subcore_barrier()

    @pl.when(sid < FROW // 16)
    def _():
        pltpu.sync_copy(acc.at[pl.ds(sid * 16, 16)],
                        out_hbm.at[cid, pl.ds(sid * 16, 16)])


def _dinv(deg_ref):
    return lax.rsqrt(deg_ref[0] + deg_ref[1] + 1.0)


def _tc_scale_in(x_ref, w1_ref, deg_ref, hs_ref):
    dinv = _dinv(deg_ref)
    h = jnp.dot(x_ref[...], w1_ref[...], preferred_element_type=jnp.float32)
    hs_ref[...] = h * dinv


def _tc_mid(p_ref, hs_ref, deg_ref, b1_ref, w2_ref, h2s_ref):
    dinv = _dinv(deg_ref)
    t1 = jnp.tanh((p_ref[0] + p_ref[1] + hs_ref[...]) * dinv + b1_ref[...])
    h2 = jnp.dot(t1, w2_ref[...], preferred_element_type=jnp.float32)
    h2s_ref[...] = h2 * dinv


def _tc_head(q_ref, h2s_ref, deg_ref, b2_ref, wc_ref, bc_ref, out_ref):
    dinv = _dinv(deg_ref)
    emb = jnp.tanh((q_ref[0] + q_ref[1] + h2s_ref[...]) * dinv + b2_ref[...])
    o = jnp.dot(emb, wc_ref[...], preferred_element_type=jnp.float32)
    out_ref[...] = jax.nn.sigmoid(o + bc_ref[...])


def _row_spec(d):
    return pl.BlockSpec((BLK, d), lambda i: (i, 0))


def _full_spec(shape):
    return pl.BlockSpec(shape, lambda i: (0,) * len(shape))


_deg_spec = pl.BlockSpec((NC, BLK, 1), lambda i: (0, i, 0))


def kernel(x, edge_index, W1, b1, W2, b2, Wc, bc):
    f32 = jnp.float32
    src = edge_index[0].astype(jnp.int32)
    dst = edge_index[1].astype(jnp.int32)
    b1r = b1.reshape(1, 128)
    b2r = b2.reshape(1, 2)
    bcr = bc.reshape(1, 1)

    deg = _deg_kernel(dst).reshape(NC, NP, 1)

    hs = pl.pallas_call(
        _tc_scale_in,
        grid=(GRID,),
        in_specs=[_row_spec(165), _full_spec((165, 128)), _deg_spec],
        out_specs=_row_spec(128),
        out_shape=jax.ShapeDtypeStruct((N, 128), f32),
    )(x, W1, deg)

    p = _agg128(hs, src, dst)

    h2s = pl.pallas_call(
        _tc_mid,
        grid=(GRID,),
        in_specs=[pl.BlockSpec((NC, BLK, 128), lambda i: (0, i, 0)),
                  _row_spec(128), _deg_spec,
                  _full_spec((1, 128)), _full_spec((128, 2))],
        out_specs=_row_spec(2),
        out_shape=jax.ShapeDtypeStruct((N, 2), f32),
    )(p, hs, deg, b1r, W2)

    q = _agg2(h2s.reshape(N * 2), src, dst).reshape(NC, NP, 2)

    out = pl.pallas_call(
        _tc_head,
        grid=(GRID,),
        in_specs=[pl.BlockSpec((NC, BLK, 2), lambda i: (0, i, 0)),
                  _row_spec(2), _deg_spec,
                  _full_spec((1, 2)), _full_spec((2, 1)), _full_spec((1, 1))],
        out_specs=_row_spec(1),
        out_shape=jax.ShapeDtypeStruct((N, 1), f32),
    )(q, h2s, deg, b2r, Wc, bcr)

    return out


# speedup vs baseline: 36.9945x; 1.2450x over previous
import functools

import jax
import jax.numpy as jnp
from jax import lax
from jax.experimental import pallas as pl
from jax.experimental.pallas import tpu as pltpu
from jax.experimental.pallas import tpu_sc as plsc

N = 10000
NP = 10240
E = 320000
NC, NS = 2, 16
NW = NC * NS
EPT = E // NW
CHUNK = 80
NCHUNK = EPT // CHUNK
BLK = 1000
GRID = N // BLK
DROW = NP // 128
FROW = NP * 2 // 128

_mesh = plsc.VectorSubcoreMesh(core_axis_name="c", subcore_axis_name="s",
                               num_cores=NC, num_subcores=NS)
_sc_params = pltpu.CompilerParams(needs_layout_passes=False)


def _zero_fill(ref, nrows, ncols):
    z = jnp.zeros((16,), jnp.float32)
    per_row = ncols // 16

    def body(i, _):
        ref[i // per_row, pl.ds((i % per_row) * 16, 16)] = z
        return 0

    lax.fori_loop(0, nrows * per_row, body, 0)


def _zero_fill1(ref, n):
    z = jnp.zeros((16,), jnp.float32)

    def body(i, _):
        ref[pl.ds(i * 16, 16)] = z
        return 0

    lax.fori_loop(0, n // 16, body, 0)


def _fill_iota_row(ref, n):
    base = lax.iota(jnp.int32, 16)
    for k in range(n // 16):
        ref[0, pl.ds(k * 16, 16)] = base + (k * 16)


def _repack(flat_ref, mat_ref, nrows):
    def body(r, _):
        for k in range(8):
            mat_ref[r, pl.ds(k * 16, 16)] = flat_ref[pl.ds(r * 128 + k * 16, 16)]
        return 0

    lax.fori_loop(0, nrows, body, 0)


@functools.partial(
    pl.kernel,
    out_type=jax.ShapeDtypeStruct((NC, DROW, 128), jnp.float32),
    mesh=_mesh,
    compiler_params=_sc_params,
    scratch_types=[
        pltpu.VMEM((EPT,), jnp.int32),
        pltpu.VMEM((NP,), jnp.float32),
        pltpu.VMEM((DROW, 128), jnp.float32),
        pltpu.VMEM((1, DROW), jnp.int32),
        pltpu.VMEM_SHARED((DROW, 128), jnp.float32),
    ],
)
def _deg_kernel(dst_hbm, out_hbm, dst_v, hist_v, mat_v, id_v, acc):
    cid = lax.axis_index("c")
    sid = lax.axis_index("s")
    wid = cid * NS + sid

    pltpu.sync_copy(dst_hbm.at[pl.ds(wid * EPT, EPT)], dst_v)
    _zero_fill1(hist_v, NP)
    _zero_fill(mat_v, DROW, 128)
    _fill_iota_row(id_v, DROW)

    @pl.when(sid < DROW // 16)
    def _():
        pltpu.sync_copy(mat_v.at[pl.ds(sid * 16, 16)],
                        acc.at[pl.ds(sid * 16, 16)])
    plsc.subcore_barrier()

    one = jnp.ones((16,), jnp.float32)

    def body(i, _):
        plsc.addupdate_scatter(hist_v, [dst_v[pl.ds(i * 16, 16)]], one)
        return 0

    lax.fori_loop(0, EPT // 16, body, 0)
    _repack(hist_v, mat_v, DROW)
    pltpu.sync_copy(mat_v, acc.at[id_v.at[0]], add=True)
    plsc.subcore_barrier()

    @pl.when(sid < DROW // 16)
    def _():
        pltpu.sync_copy(acc.at[pl.ds(sid * 16, 16)],
                        out_hbm.at[cid, pl.ds(sid * 16, 16)])


@functools.partial(
    pl.kernel,
    out_type=jax.ShapeDtypeStruct((NC, N, 128), jnp.float32),
    mesh=_mesh,
    compiler_params=_sc_params,
    scratch_types=[
        pltpu.VMEM((EPT,), jnp.int32),
        pltpu.VMEM((3, CHUNK), jnp.int32),
        pltpu.VMEM((3, CHUNK, 128), jnp.float32),
        pltpu.VMEM_SHARED((NP, 128), jnp.float32),
        pltpu.SemaphoreType.DMA((3,)),
        pltpu.SemaphoreType.DMA((3,)),
        pltpu.SemaphoreType.DMA((3,)),
    ],
)
def _agg128(table_hbm, src_hbm, dst_hbm, out_hbm,
            src_v, dst_v, rows_v, acc, gsem, ssem, dsem):
    cid = lax.axis_index("c")
    sid = lax.axis_index("s")
    wid = cid * NS + sid
    ebase = wid * EPT

    pltpu.sync_copy(src_hbm.at[pl.ds(ebase, EPT)], src_v)
    for b in range(3):
        _zero_fill(rows_v.at[b], CHUNK, 128)
    for k in range(640 // CHUNK):
        pltpu.sync_copy(rows_v.at[0], acc.at[pl.ds(sid * 640 + k * CHUNK, CHUNK)])
    plsc.subcore_barrier()

    def dload(j, b):
        return pltpu.make_async_copy(
            dst_hbm.at[pl.ds(ebase + j * CHUNK, CHUNK)], dst_v.at[b], dsem.at[b])

    def gath(j, b):
        return pltpu.make_async_copy(
            table_hbm.at[src_v.at[pl.ds(j * CHUNK, CHUNK)]], rows_v.at[b],
            gsem.at[b])

    def scat(b):
        return pltpu.make_async_copy(
            rows_v.at[b], acc.at[dst_v.at[b]], ssem.at[b])

    for b in range(2):
        dload(b, b).start()
        gath(b, b).start()

    def body(j, _):
        b = lax.rem(j, 3)
        nb = lax.rem(j + 2, 3)
        gath(j, b).wait()

        @pl.when(j >= 1)
        def _():
            scat(nb).wait()

        @pl.when(j + 2 < NCHUNK)
        def _():
            dload(j + 2, nb).start()
            gath(j + 2, nb).start()

        dload(j, b).wait()
        pltpu.async_copy(rows_v.at[b], acc.at[dst_v.at[b]], ssem.at[b],
                         add=True)
        return 0

    lax.fori_loop(0, NCHUNK, body, 0)
    scat((NCHUNK - 1) % 3).wait()
    plsc.subcore_barrier()

    @pl.when(sid < 10)
    def _():
        pltpu.sync_copy(acc.at[pl.ds(sid * BLK, BLK)],
                        out_hbm.at[cid, pl.ds(sid * BLK, BLK)])


@functools.partial(
    pl.kernel,
    out_type=jax.ShapeDtypeStruct((NC, FROW, 128), jnp.float32),
    mesh=_mesh,
    compiler_params=_sc_params,
    scratch_types=[
        pltpu.VMEM((EPT,), jnp.int32),
        pltpu.VMEM((EPT,), jnp.int32),
        pltpu.VMEM((N * 2,), jnp.float32),
        pltpu.VMEM((NP * 2,), jnp.float32),
        pltpu.VMEM((FROW, 128), jnp.float32),
        pltpu.VMEM((1, FROW), jnp.int32),
        pltpu.VMEM_SHARED((FROW, 128), jnp.float32),
        pltpu.SemaphoreType.DMA((3,)),
    ],
)
def _agg2(table_hbm, src_hbm, dst_hbm, out_hbm,
          src_v, dst_v, tab_v, acc_v, mat_v, id_v, acc, psem):
    cid = lax.axis_index("c")
    sid = lax.axis_index("s")
    wid = cid * NS + sid

    cp_s = pltpu.make_async_copy(src_hbm.at[pl.ds(wid * EPT, EPT)], src_v,
                                 psem.at[0])
    cp_d = pltpu.make_async_copy(dst_hbm.at[pl.ds(wid * EPT, EPT)], dst_v,
                                 psem.at[1])
    cp_t = pltpu.make_async_copy(table_hbm, tab_v, psem.at[2])
    cp_s.start()
    cp_d.start()
    cp_t.start()
    _zero_fill1(acc_v, NP * 2)
    _zero_fill(mat_v, FROW, 128)
    _fill_iota_row(id_v, FROW)

    @pl.when(sid < FROW // 16)
    def _():
        pltpu.sync_copy(mat_v.at[pl.ds(sid * 16, 16)],
                        acc.at[pl.ds(sid * 16, 16)])
    cp_s.wait()
    cp_d.wait()
    cp_t.wait()
    plsc.subcore_barrier()

    def body(i, _):
        s2 = src_v[pl.ds(i * 16, 16)] * 2
        d2 = dst_v[pl.ds(i * 16, 16)] * 2
        for c in range(2):
            v = plsc.load_gather(tab_v, [s2 + c])
            plsc.addupdate_scatter(acc_v, [d2 + c], v)
        return 0

    lax.fori_loop(0, EPT // 16, body, 0)
    _repack(acc_v, mat_v, FROW)
    pltpu.sync_copy(mat_v, acc.at[id_v.at[0]], add=True)
    plsc.subcore_barrier()

    @pl.when(sid < FROW // 16)
    def _():
        pltpu.sync_copy(acc.at[pl.ds(sid * 16, 16)],
                        out_hbm.at[cid, pl.ds(sid * 16, 16)])


def _dinv(deg_ref):
    return lax.rsqrt(deg_ref[0] + deg_ref[1] + 1.0)


def _tc_scale_in(x_ref, w1_ref, deg_ref, hs_ref):
    dinv = _dinv(deg_ref)
    h = jnp.dot(x_ref[...], w1_ref[...], preferred_element_type=jnp.float32)
    hs_ref[...] = h * dinv


def _tc_mid(p_ref, hs_ref, deg_ref, b1_ref, w2_ref, h2s_ref):
    dinv = _dinv(deg_ref)
    t1 = jnp.tanh((p_ref[0] + p_ref[1] + hs_ref[...]) * dinv + b1_ref[...])
    h2 = jnp.dot(t1, w2_ref[...], preferred_element_type=jnp.float32)
    h2s_ref[...] = h2 * dinv


def _tc_head(q_ref, h2s_ref, deg_ref, b2_ref, wc_ref, bc_ref, out_ref):
    dinv = _dinv(deg_ref)
    emb = jnp.tanh((q_ref[0] + q_ref[1] + h2s_ref[...]) * dinv + b2_ref[...])
    o = jnp.dot(emb, wc_ref[...], preferred_element_type=jnp.float32)
    out_ref[...] = jax.nn.sigmoid(o + bc_ref[...])


def _row_spec(d):
    return pl.BlockSpec((BLK, d), lambda i: (i, 0))


def _full_spec(shape):
    return pl.BlockSpec(shape, lambda i: (0,) * len(shape))


_deg_spec = pl.BlockSpec((NC, BLK, 1), lambda i: (0, i, 0))


def kernel(x, edge_index, W1, b1, W2, b2, Wc, bc):
    f32 = jnp.float32
    src = edge_index[0].astype(jnp.int32)
    dst = edge_index[1].astype(jnp.int32)
    b1r = b1.reshape(1, 128)
    b2r = b2.reshape(1, 2)
    bcr = bc.reshape(1, 1)

    deg = _deg_kernel(dst).reshape(NC, NP, 1)

    hs = pl.pallas_call(
        _tc_scale_in,
        grid=(GRID,),
        in_specs=[_row_spec(165), _full_spec((165, 128)), _deg_spec],
        out_specs=_row_spec(128),
        out_shape=jax.ShapeDtypeStruct((N, 128), f32),
    )(x, W1, deg)

    p = _agg128(hs, src, dst)

    h2s = pl.pallas_call(
        _tc_mid,
        grid=(GRID,),
        in_specs=[pl.BlockSpec((NC, BLK, 128), lambda i: (0, i, 0)),
                  _row_spec(128), _deg_spec,
                  _full_spec((1, 128)), _full_spec((128, 2))],
        out_specs=_row_spec(2),
        out_shape=jax.ShapeDtypeStruct((N, 2), f32),
    )(p, hs, deg, b1r, W2)

    q = _agg2(h2s.reshape(N * 2), src, dst).reshape(NC, NP, 2)

    out = pl.pallas_call(
        _tc_head,
        grid=(GRID,),
        in_specs=[pl.BlockSpec((NC, BLK, 2), lambda i: (0, i, 0)),
                  _row_spec(2), _deg_spec,
                  _full_spec((1, 2)), _full_spec((2, 1)), _full_spec((1, 1))],
        out_specs=_row_spec(1),
        out_shape=jax.ShapeDtypeStruct((N, 1), f32),
    )(q, h2s, deg, b2r, Wc, bcr)

    return out


# speedup vs baseline: 47.6389x; 1.2877x over previous
import functools

import jax
import jax.numpy as jnp
from jax import lax
from jax.experimental import pallas as pl
from jax.experimental.pallas import tpu as pltpu
from jax.experimental.pallas import tpu_sc as plsc

N = 10000
NP = 10240
E = 320000
NC, NS = 2, 16
NW = NC * NS
EPT = E // NW
CHUNK = 80
NCHUNK = EPT // CHUNK
BLK = 1024
GRID = NP // BLK
DROW = NP // 128
FROW = NP * 2 // 128

_mesh = plsc.VectorSubcoreMesh(core_axis_name="c", subcore_axis_name="s",
                               num_cores=NC, num_subcores=NS)
_sc_params = pltpu.CompilerParams(needs_layout_passes=False)


def _zero_fill(ref, nrows, ncols):
    z = jnp.zeros((16,), jnp.float32)
    per_row = ncols // 16

    def body(i, _):
        ref[i // per_row, pl.ds((i % per_row) * 16, 16)] = z
        return 0

    lax.fori_loop(0, nrows * per_row, body, 0)


def _zero_fill1(ref, n):
    z = jnp.zeros((16,), jnp.float32)

    def body(i, _):
        ref[pl.ds(i * 16, 16)] = z
        return 0

    lax.fori_loop(0, n // 16, body, 0)


def _fill_iota_row(ref, n):
    base = lax.iota(jnp.int32, 16)
    for k in range(n // 16):
        ref[0, pl.ds(k * 16, 16)] = base + (k * 16)


def _repack(flat_ref, mat_ref, nrows):
    def body(r, _):
        for k in range(8):
            mat_ref[r, pl.ds(k * 16, 16)] = flat_ref[pl.ds(r * 128 + k * 16, 16)]
        return 0

    lax.fori_loop(0, nrows, body, 0)


@functools.partial(
    pl.kernel,
    out_type=jax.ShapeDtypeStruct((NC, DROW, 128), jnp.float32),
    mesh=_mesh,
    compiler_params=_sc_params,
    scratch_types=[
        pltpu.VMEM((EPT,), jnp.int32),
        pltpu.VMEM((DROW, 128), jnp.float32),
        pltpu.VMEM((1, DROW), jnp.int32),
        pltpu.VMEM_SHARED((DROW, 128), jnp.float32),
    ],
)
def _deg_kernel(dst_hbm, out_hbm, dst_v, hist_v, id_v, acc):
    cid = lax.axis_index("c")
    sid = lax.axis_index("s")
    wid = cid * NS + sid

    pltpu.sync_copy(dst_hbm.at[pl.ds(wid * EPT, EPT)], dst_v)
    _zero_fill(hist_v, DROW, 128)
    _fill_iota_row(id_v, DROW)

    @pl.when(sid < DROW // 16)
    def _():
        pltpu.sync_copy(hist_v.at[pl.ds(sid * 16, 16)],
                        acc.at[pl.ds(sid * 16, 16)])
    plsc.subcore_barrier()

    one = jnp.ones((16,), jnp.float32)

    def body(i, _):
        d = dst_v[pl.ds(i * 16, 16)]
        plsc.addupdate_scatter(hist_v, [d >> 7, d & 127], one)
        return 0

    lax.fori_loop(0, EPT // 16, body, 0)
    pltpu.sync_copy(hist_v, acc.at[id_v.at[0]], add=True)
    plsc.subcore_barrier()

    @pl.when(sid < DROW // 16)
    def _():
        pltpu.sync_copy(acc.at[pl.ds(sid * 16, 16)],
                        out_hbm.at[cid, pl.ds(sid * 16, 16)])


@functools.partial(
    pl.kernel,
    out_type=jax.ShapeDtypeStruct((NC, NP, 128), jnp.float32),
    mesh=_mesh,
    compiler_params=_sc_params,
    scratch_types=[
        pltpu.VMEM((EPT,), jnp.int32),
        pltpu.VMEM((3, CHUNK), jnp.int32),
        pltpu.VMEM((3, CHUNK, 128), jnp.float32),
        pltpu.VMEM_SHARED((NP, 128), jnp.float32),
        pltpu.SemaphoreType.DMA((3,)),
        pltpu.SemaphoreType.DMA((3,)),
        pltpu.SemaphoreType.DMA((3,)),
    ],
)
def _agg128(table_hbm, src_hbm, dst_hbm, out_hbm,
            src_v, dst_v, rows_v, acc, gsem, ssem, dsem):
    cid = lax.axis_index("c")
    sid = lax.axis_index("s")
    wid = cid * NS + sid
    ebase = wid * EPT

    pltpu.sync_copy(src_hbm.at[pl.ds(ebase, EPT)], src_v)
    for b in range(3):
        _zero_fill(rows_v.at[b], CHUNK, 128)
    for k in range(640 // CHUNK):
        pltpu.sync_copy(rows_v.at[0], acc.at[pl.ds(sid * 640 + k * CHUNK, CHUNK)])
    plsc.subcore_barrier()

    def dload(j, b):
        return pltpu.make_async_copy(
            dst_hbm.at[pl.ds(ebase + j * CHUNK, CHUNK)], dst_v.at[b], dsem.at[b])

    def gath(j, b):
        return pltpu.make_async_copy(
            table_hbm.at[src_v.at[pl.ds(j * CHUNK, CHUNK)]], rows_v.at[b],
            gsem.at[b])

    def scat(b):
        return pltpu.make_async_copy(
            rows_v.at[b], acc.at[dst_v.at[b]], ssem.at[b])

    for b in range(2):
        dload(b, b).start()
        gath(b, b).start()

    def body(j, _):
        b = lax.rem(j, 3)
        nb = lax.rem(j + 2, 3)
        gath(j, b).wait()

        @pl.when(j >= 1)
        def _():
            scat(nb).wait()

        @pl.when(j + 2 < NCHUNK)
        def _():
            dload(j + 2, nb).start()
            gath(j + 2, nb).start()

        dload(j, b).wait()
        pltpu.async_copy(rows_v.at[b], acc.at[dst_v.at[b]], ssem.at[b],
                         add=True)
        return 0

    lax.fori_loop(0, NCHUNK, body, 0)
    scat((NCHUNK - 1) % 3).wait()
    plsc.subcore_barrier()

    pltpu.sync_copy(acc.at[pl.ds(sid * 640, 640)],
                    out_hbm.at[cid, pl.ds(sid * 640, 640)])


@functools.partial(
    pl.kernel,
    out_type=jax.ShapeDtypeStruct((NC, FROW, 128), jnp.float32),
    mesh=_mesh,
    compiler_params=_sc_params,
    scratch_types=[
        pltpu.VMEM((EPT,), jnp.int32),
        pltpu.VMEM((EPT,), jnp.int32),
        pltpu.VMEM((NP * 2,), jnp.float32),
        pltpu.VMEM((FROW, 128), jnp.float32),
        pltpu.VMEM((1, FROW), jnp.int32),
        pltpu.VMEM_SHARED((FROW, 128), jnp.float32),
        pltpu.SemaphoreType.DMA((3,)),
    ],
)
def _agg2(table_hbm, src_hbm, dst_hbm, out_hbm,
          src_v, dst_v, tab_v, acc_v, id_v, acc, psem):
    cid = lax.axis_index("c")
    sid = lax.axis_index("s")
    wid = cid * NS + sid

    cp_s = pltpu.make_async_copy(src_hbm.at[pl.ds(wid * EPT, EPT)], src_v,
                                 psem.at[0])
    cp_d = pltpu.make_async_copy(dst_hbm.at[pl.ds(wid * EPT, EPT)], dst_v,
                                 psem.at[1])
    cp_t = pltpu.make_async_copy(table_hbm, tab_v, psem.at[2])
    cp_s.start()
    cp_d.start()
    cp_t.start()
    _zero_fill(acc_v, FROW, 128)
    _fill_iota_row(id_v, FROW)

    @pl.when(sid < FROW // 16)
    def _():
        pltpu.sync_copy(acc_v.at[pl.ds(sid * 16, 16)],
                        acc.at[pl.ds(sid * 16, 16)])
    cp_s.wait()
    cp_d.wait()
    cp_t.wait()
    plsc.subcore_barrier()

    def body(i, _):
        sv = src_v[pl.ds(i * 16, 16)]
        dv = dst_v[pl.ds(i * 16, 16)]
        for c in range(2):
            v = plsc.load_gather(tab_v, [sv + c * NP])
            d = dv + c * NP
            plsc.addupdate_scatter(acc_v, [d >> 7, d & 127], v)
        return 0

    lax.fori_loop(0, EPT // 16, body, 0)
    pltpu.sync_copy(acc_v, acc.at[id_v.at[0]], add=True)
    plsc.subcore_barrier()

    @pl.when(sid < FROW // 16)
    def _():
        pltpu.sync_copy(acc.at[pl.ds(sid * 16, 16)],
                        out_hbm.at[cid, pl.ds(sid * 16, 16)])


def _dinv_grid(deg_ref):
    return lax.rsqrt(deg_ref[0] + deg_ref[1] + 1.0)


def _tc_scale_in(xt_ref, w1_ref, deg_ref, hs_ref):
    dinv3 = _dinv_grid(deg_ref)[:, :, None]
    h = lax.dot_general(xt_ref[...], w1_ref[...], (((0,), (0,)), ((), ())),
                        preferred_element_type=jnp.float32)
    hs_ref[...] = (h.reshape(8, 128, 128) * dinv3).reshape(BLK, 128)


def _tc_mid(p_ref, hs_ref, deg_ref, b1_ref, w2_ref, h2s_ref):
    dinv3 = _dinv_grid(deg_ref)[:, :, None]
    s3 = (p_ref[0] + p_ref[1] + hs_ref[...]).reshape(8, 128, 128)
    t1 = jnp.tanh(s3 * dinv3 + b1_ref[...][None]).reshape(BLK, 128)
    h2 = jnp.dot(t1, w2_ref[...], preferred_element_type=jnp.float32)
    h2t = (h2.T.reshape(2, 8, 128) * _dinv_grid(deg_ref)[None])
    h2s_ref[...] = h2t


def _tc_head(q_ref, h2s_ref, deg_ref, b2_ref, wc_ref, bc_ref, out_ref):
    dinv = _dinv_grid(deg_ref)
    qs = q_ref[0] + q_ref[1] + h2s_ref[...]
    emb = jnp.tanh(qs * dinv[None] + b2_ref[...][:, :, None])
    o = emb[0] * wc_ref[0, 0] + emb[1] * wc_ref[1, 0] + bc_ref[0, 0]
    out_ref[...] = jax.nn.sigmoid(o)


def _full_spec(shape):
    return pl.BlockSpec(shape, lambda i: (0,) * len(shape))


_deg_spec = pl.BlockSpec((NC, 8, 128), lambda i: (0, i, 0))


def kernel(x, edge_index, W1, b1, W2, b2, Wc, bc):
    f32 = jnp.float32
    src = edge_index[0].astype(jnp.int32)
    dst = edge_index[1].astype(jnp.int32)
    b1r = b1.reshape(1, 128)
    b2r = b2.reshape(2, 1)
    bcr = bc.reshape(1, 1)

    deg = _deg_kernel(dst)

    hs = pl.pallas_call(
        _tc_scale_in,
        grid=(GRID,),
        in_specs=[pl.BlockSpec((165, BLK), lambda i: (0, i)),
                  _full_spec((165, 128)), _deg_spec],
        out_specs=pl.BlockSpec((BLK, 128), lambda i: (i, 0)),
        out_shape=jax.ShapeDtypeStruct((NP, 128), f32),
    )(x.T, W1, deg)

    p = _agg128(hs, src, dst)

    h2st = pl.pallas_call(
        _tc_mid,
        grid=(GRID,),
        in_specs=[pl.BlockSpec((NC, BLK, 128), lambda i: (0, i, 0)),
                  pl.BlockSpec((BLK, 128), lambda i: (i, 0)), _deg_spec,
                  _full_spec((1, 128)), _full_spec((128, 2))],
        out_specs=pl.BlockSpec((2, 8, 128), lambda i: (0, i, 0)),
        out_shape=jax.ShapeDtypeStruct((2, DROW, 128), f32),
    )(p, hs, deg, b1r, W2)

    q = _agg2(h2st.reshape(NP * 2), src, dst)
    q4 = q.reshape(NC, 2, DROW, 128)

    out = pl.pallas_call(
        _tc_head,
        grid=(GRID,),
        in_specs=[pl.BlockSpec((NC, 2, 8, 128), lambda i: (0, 0, i, 0)),
                  pl.BlockSpec((2, 8, 128), lambda i: (0, i, 0)), _deg_spec,
                  _full_spec((2, 1)), _full_spec((2, 1)), _full_spec((1, 1))],
        out_specs=pl.BlockSpec((8, 128), lambda i: (i, 0)),
        out_shape=jax.ShapeDtypeStruct((DROW, 128), f32),
    )(q4, h2st, deg, b2r, Wc, bcr)

    return out.reshape(NP)[:N].reshape(N, 1)
